# Initial kernel scaffold; baseline (speedup 1.0000x reference)
import jax
import jax.numpy as jnp
from jax.experimental import pallas as pl


def kernel(hidden_states, W0, b0, gate_W, u, svh, expert_bias):
    raise NotImplementedError("write your pallas kernel here")


# speedup vs baseline: 2.0875x; 2.0875x over previous
import jax
import jax.numpy as jnp
from jax.experimental import pallas as pl

E = 8
GATE_K = 16
K = 256
D_IN = 2048
D_OUT = 2048
T = 128


def _body(x_ref, w0_ref, b0_ref, gw_ref, sel_ref, svh_ref, u_ref, eb_ref,
          r8_ref, o_ref):
    x = x_ref[...]
    g = jnp.dot(x, gw_ref[...])
    n2 = jnp.dot(g * g, sel_ref[...],
                 precision=jax.lax.Precision.HIGHEST)
    logit = jnp.sqrt(n2)
    ii = jax.lax.broadcasted_iota(jnp.int32, (T, E), 1)
    m1 = jnp.max(logit, axis=1, keepdims=True)
    i1 = jnp.min(jnp.where(logit >= m1, ii, E), axis=1, keepdims=True)
    l2 = jnp.where(ii == i1, jnp.float32(-1e30), logit)
    m2 = jnp.max(l2, axis=1, keepdims=True)
    i2 = jnp.min(jnp.where(l2 >= m2, ii, E), axis=1, keepdims=True)
    w1 = 1.0 / (1.0 + jnp.exp(m2 - m1))
    wf = jnp.where(ii == i1, w1, 0.0) + jnp.where(ii == i2, 1.0 - w1, 0.0)

    pret = jnp.dot(x, w0_ref[...]) + b0_ref[...]
    z = jnp.dot(x, svh_ref[...])
    wcols = jnp.dot(wf, r8_ref[...])
    y = jnp.dot(z * wcols, u_ref[...]) + jnp.dot(wf, eb_ref[...])
    o_ref[...] = pret + y


def kernel(hidden_states, W0, b0, gate_W, u, svh, expert_bias):
    Bb, Ss, Dd = hidden_states.shape
    n = Bb * Ss
    hs = hidden_states.reshape(n, D_IN)
    W0T = W0.T
    gWT = gate_W.T
    SVHT = svh.reshape(E * K, D_IN).T
    U_all = u.transpose(0, 2, 1).reshape(E * K, D_OUT)
    b0r = b0.reshape(1, D_OUT)
    Sel = (jnp.arange(E * GATE_K)[:, None] // GATE_K
           == jnp.arange(E)[None, :]).astype(jnp.float32)
    R8 = (jnp.arange(E)[:, None]
          == jnp.arange(E * K)[None, :] // K).astype(jnp.float32)

    out = pl.pallas_call(
        _body,
        grid=(n // T,),
        in_specs=[
            pl.BlockSpec((T, D_IN), lambda i: (i, 0)),
            pl.BlockSpec((D_IN, D_OUT), lambda i: (0, 0)),
            pl.BlockSpec((1, D_OUT), lambda i: (0, 0)),
            pl.BlockSpec((D_IN, E * GATE_K), lambda i: (0, 0)),
            pl.BlockSpec((E * GATE_K, E), lambda i: (0, 0)),
            pl.BlockSpec((D_IN, E * K), lambda i: (0, 0)),
            pl.BlockSpec((E * K, D_OUT), lambda i: (0, 0)),
            pl.BlockSpec((E, D_OUT), lambda i: (0, 0)),
            pl.BlockSpec((E, E * K), lambda i: (0, 0)),
        ],
        out_specs=pl.BlockSpec((T, D_OUT), lambda i: (i, 0)),
        out_shape=jax.ShapeDtypeStruct((n, D_OUT), jnp.float32),
    )(hs, W0T, b0r, gWT, Sel, SVHT, U_all, expert_bias, R8)
    return out.reshape(Bb, Ss, D_OUT)
